# baseline (speedup 1.0000x reference)
import functools

import jax
import jax.numpy as jnp
from jax import lax
from jax.experimental import pallas as pl
from jax.experimental.pallas import tpu as pltpu
from jax.experimental.pallas import tpu_sc as plsc

N = 10000
D = 128
HD = D // 2
E = 320000
NC = 2
NS = 16
CH = 128
CPT = 160
PAIRS = CPT // 2
NCHT = NS * CPT
PE = NCHT * CH
NP = 10240
TRASH = NP - 1
RPT = NP // NS
ZCH = 128
CW = 16


def _agg_body(with_cnt, x2_hbm, src_hbm, dst_hbm, *refs):
    if with_cnt:
        (agg_hbm, cnt_hbm, acc_sh, cnt_sh, rows_a, rows_b, src_v, dst_v,
         ones_v, zc_v, sem_a, sem_b) = refs
    else:
        (agg_hbm, acc_sh, rows_a, rows_b, src_v, dst_v,
         sem_a, sem_b) = refs
    c = lax.axis_index("c")
    s = lax.axis_index("s")
    cbase = s * CPT

    z16 = jnp.zeros((16,), jnp.float32)
    one16 = jnp.ones((16,), jnp.float32)

    def _zero_rows(r, _):
        for l in range(HD // 16):
            rows_a[r, pl.ds(l * 16, 16)] = z16
        return 0
    lax.fori_loop(0, ZCH, _zero_rows, 0)

    for q in range(RPT // ZCH):
        pltpu.sync_copy(rows_a.at[pl.ds(0, ZCH)],
                        acc_sh.at[pl.ds(s * RPT + q * ZCH, ZCH)])

    if with_cnt:
        def _zero_cnt(r, _):
            zc_v[r, pl.ds(0, 16)] = z16
            return 0
        lax.fori_loop(0, RPT, _zero_cnt, 0)

        def _fill_ones(r, _):
            ones_v[r, pl.ds(0, 16)] = one16
            return 0
        lax.fori_loop(0, CH, _fill_ones, 0)

        pltpu.sync_copy(zc_v, cnt_sh.at[pl.ds(s * RPT, RPT)])

    pltpu.sync_copy(src_hbm.at[pl.ds(cbase, CPT)], src_v)
    pltpu.sync_copy(dst_hbm.at[pl.ds(cbase, CPT)], dst_v)

    def _remap(r, _):
        for l in range(CH // 16):
            v = src_v[r, pl.ds(l * 16, 16)]
            src_v[r, pl.ds(l * 16, 16)] = v + v + c
        return 0
    lax.fori_loop(0, CPT, _remap, 0)

    plsc.subcore_barrier()

    pltpu.async_copy(x2_hbm.at[src_v.at[0]], rows_a, sem_a)
    pltpu.async_copy(x2_hbm.at[src_v.at[1]], rows_b, sem_b)

    def _half(k, ck, rows, sem, cnt_core):
        pltpu.make_async_copy(x2_hbm.at[src_v.at[ck]], rows, sem).wait()
        pltpu.sync_copy(rows, acc_sh.at[dst_v.at[ck]], add=True)

        @pl.when(k < PAIRS - 1)
        def _():
            pltpu.async_copy(x2_hbm.at[src_v.at[ck + 2]], rows, sem)
        if with_cnt:
            @pl.when(c == cnt_core)
            def _():
                pltpu.sync_copy(ones_v, cnt_sh.at[dst_v.at[ck]], add=True)

    def _edge_pair(k, _):
        _half(k, 2 * k, rows_a, sem_a, 0)
        _half(k, 2 * k + 1, rows_b, sem_b, 1)
        return 0
    lax.fori_loop(0, PAIRS, _edge_pair, 0)

    plsc.subcore_barrier()

    pltpu.sync_copy(acc_sh.at[pl.ds(s * RPT, RPT)],
                    agg_hbm.at[c, pl.ds(s * RPT, RPT)])
    if with_cnt:
        pltpu.sync_copy(cnt_sh.at[pl.ds(s * RPT, RPT)],
                        cnt_hbm.at[c, pl.ds(s * RPT, RPT)])


@functools.lru_cache(maxsize=None)
def _make_sc_aggregate(with_cnt):
    out_type = [jax.ShapeDtypeStruct((NC, NP, HD), jnp.float32)]
    scratch = [
        pltpu.VMEM_SHARED((NP, HD), jnp.float32),
        pltpu.VMEM((CH, HD), jnp.float32),
        pltpu.VMEM((CH, HD), jnp.float32),
        pltpu.VMEM((CPT, CH), jnp.int32),
        pltpu.VMEM((CPT, CH), jnp.int32),
        pltpu.SemaphoreType.DMA,
        pltpu.SemaphoreType.DMA,
    ]
    if with_cnt:
        out_type.append(jax.ShapeDtypeStruct((NC, NP, CW), jnp.float32))
        scratch[1:1] = [pltpu.VMEM_SHARED((NP, CW), jnp.float32)]
        scratch[6:6] = [pltpu.VMEM((CH, CW), jnp.float32),
                        pltpu.VMEM((RPT, CW), jnp.float32)]

    @functools.partial(
        pl.kernel,
        out_type=tuple(out_type),
        mesh=plsc.VectorSubcoreMesh(core_axis_name="c", subcore_axis_name="s",
                                    num_cores=NC, num_subcores=NS),
        scratch_types=tuple(scratch),
        compiler_params=pltpu.CompilerParams(use_tc_tiling_on_sc=False),
    )
    def _sc_aggregate(*refs):
        _agg_body(with_cnt, *refs)

    return _sc_aggregate


BR = 1000


def _combine_body(relu, x_ref, agg_ref, cnt_ref, wl_ref, wr_ref, b_ref,
                  o_ref):
    cnt = cnt_ref[0, :, 0:1] + cnt_ref[1, :, 0:1]
    inv = 1.0 / jnp.maximum(cnt, 1.0)
    mean = jnp.concatenate([agg_ref[0], agg_ref[1]], axis=1) * inv
    acc = (jnp.dot(mean, wl_ref[...], preferred_element_type=jnp.float32)
           + jnp.dot(x_ref[...], wr_ref[...], preferred_element_type=jnp.float32)
           + b_ref[...])
    o_ref[...] = jnp.maximum(acc, 0.0) if relu else acc


def _tc_combine(x, agg, cnt, Wl, Wr, b, relu):
    return pl.pallas_call(
        functools.partial(_combine_body, relu),
        grid=(N // BR,),
        in_specs=[
            pl.BlockSpec((BR, D), lambda i: (i, 0)),
            pl.BlockSpec((NC, BR, HD), lambda i: (0, i, 0)),
            pl.BlockSpec((NC, BR, CW), lambda i: (0, i, 0)),
            pl.BlockSpec((D, D), lambda i: (0, 0)),
            pl.BlockSpec((D, D), lambda i: (0, 0)),
            pl.BlockSpec((1, D), lambda i: (0, 0)),
        ],
        out_specs=pl.BlockSpec((BR, D), lambda i: (i, 0)),
        out_shape=jax.ShapeDtypeStruct((N, D), jnp.float32),
    )(x, agg, cnt, Wl, Wr, b.reshape(1, D))


def kernel(x, edge_index, Wl1, Wr1, b1, Wl2, Wr2, b2):
    src = edge_index[0].astype(jnp.int32)
    dst = edge_index[1].astype(jnp.int32)
    srcp = jnp.concatenate(
        [src, jnp.zeros((PE - E,), jnp.int32)]).reshape(NCHT, CH)
    dstp = jnp.concatenate(
        [dst, jnp.full((PE - E,), TRASH, jnp.int32)]).reshape(NCHT, CH)
    agg1, cnt = _make_sc_aggregate(True)(x.reshape(2 * N, HD), srcp, dstp)
    h = _tc_combine(x, agg1, cnt, Wl1, Wr1, b1, relu=True)
    agg2, = _make_sc_aggregate(False)(h.reshape(2 * N, HD), srcp, dstp)
    return _tc_combine(h, agg2, cnt, Wl2, Wr2, b2, relu=False)


# speedup vs baseline: 1.0677x; 1.0677x over previous
import functools

import jax
import jax.numpy as jnp
from jax import lax
from jax.experimental import pallas as pl
from jax.experimental.pallas import tpu as pltpu
from jax.experimental.pallas import tpu_sc as plsc

N = 10000
D = 128
HD = D // 2
E = 320000
NC = 2
NS = 16
CH = 128
CPT = 160
PAIRS = CPT // 2
NCHT = NS * CPT
PE = NCHT * CH
NP = 10240
TRASH = NP - 1
RPT = NP // NS
ZCH = 128
CW = 16


def _agg_body(with_cnt, x01_hbm, src_hbm, dst_hbm, *refs):
    if with_cnt:
        (agg_hbm, cnt_hbm, acc_sh, cnt_sh, rows_a, rows_b, src_v, dst_v,
         ones_v, zc_v, sem_a, sem_b, sem_s) = refs
    else:
        (agg_hbm, acc_sh, rows_a, rows_b, src_v, dst_v,
         sem_a, sem_b, sem_s) = refs
    c = lax.axis_index("c")
    s = lax.axis_index("s")
    cbase = s * CPT

    z16 = jnp.zeros((16,), jnp.float32)
    one16 = jnp.ones((16,), jnp.float32)

    def _zero_rows(r, _):
        for l in range(HD // 16):
            rows_a[r, pl.ds(l * 16, 16)] = z16
        return 0
    lax.fori_loop(0, ZCH, _zero_rows, 0)

    for q in range(RPT // ZCH):
        pltpu.sync_copy(rows_a.at[pl.ds(0, ZCH)],
                        acc_sh.at[pl.ds(s * RPT + q * ZCH, ZCH)])

    if with_cnt:
        def _zero_cnt(r, _):
            zc_v[r, pl.ds(0, 16)] = z16
            return 0
        lax.fori_loop(0, RPT, _zero_cnt, 0)

        def _fill_ones(r, _):
            ones_v[r, pl.ds(0, 16)] = one16
            return 0
        lax.fori_loop(0, CH, _fill_ones, 0)

        pltpu.sync_copy(zc_v, cnt_sh.at[pl.ds(s * RPT, RPT)])

    pltpu.sync_copy(src_hbm.at[pl.ds(cbase, CPT)], src_v)
    pltpu.sync_copy(dst_hbm.at[pl.ds(cbase, CPT)], dst_v)

    plsc.subcore_barrier()

    table = x01_hbm.at[c]

    pltpu.async_copy(table.at[src_v.at[0]], rows_a, sem_a)
    pltpu.async_copy(table.at[src_v.at[1]], rows_b, sem_b)

    def _edge_pair(k, _):
        ck = 2 * k
        pltpu.make_async_copy(table.at[src_v.at[ck]], rows_a, sem_a).wait()
        pltpu.async_copy(rows_a, acc_sh.at[dst_v.at[ck]], sem_s, add=True)
        if with_cnt:
            @pl.when(c == 0)
            def _():
                pltpu.sync_copy(ones_v, cnt_sh.at[dst_v.at[ck]], add=True)
        pltpu.make_async_copy(table.at[src_v.at[ck + 1]], rows_b,
                              sem_b).wait()
        pltpu.async_copy(rows_b, acc_sh.at[dst_v.at[ck + 1]], sem_s,
                         add=True)
        if with_cnt:
            @pl.when(c == 1)
            def _():
                pltpu.sync_copy(ones_v, cnt_sh.at[dst_v.at[ck + 1]],
                                add=True)
        pltpu.make_async_copy(rows_a, acc_sh.at[dst_v.at[ck]],
                              sem_s).wait()
        pltpu.make_async_copy(rows_b, acc_sh.at[dst_v.at[ck + 1]],
                              sem_s).wait()

        @pl.when(k < PAIRS - 1)
        def _():
            pltpu.async_copy(table.at[src_v.at[ck + 2]], rows_a, sem_a)
            pltpu.async_copy(table.at[src_v.at[ck + 3]], rows_b, sem_b)
        return 0
    lax.fori_loop(0, PAIRS, _edge_pair, 0)

    plsc.subcore_barrier()

    pltpu.sync_copy(acc_sh.at[pl.ds(s * RPT, RPT)],
                    agg_hbm.at[c, pl.ds(s * RPT, RPT)])
    if with_cnt:
        pltpu.sync_copy(cnt_sh.at[pl.ds(s * RPT, RPT)],
                        cnt_hbm.at[c, pl.ds(s * RPT, RPT)])


@functools.lru_cache(maxsize=None)
def _make_sc_aggregate(with_cnt):
    out_type = [jax.ShapeDtypeStruct((NC, NP, HD), jnp.float32)]
    scratch = [
        pltpu.VMEM_SHARED((NP, HD), jnp.float32),
        pltpu.VMEM((CH, HD), jnp.float32),
        pltpu.VMEM((CH, HD), jnp.float32),
        pltpu.VMEM((CPT, CH), jnp.int32),
        pltpu.VMEM((CPT, CH), jnp.int32),
        pltpu.SemaphoreType.DMA,
        pltpu.SemaphoreType.DMA,
        pltpu.SemaphoreType.DMA,
    ]
    if with_cnt:
        out_type.append(jax.ShapeDtypeStruct((NC, NP, CW), jnp.float32))
        scratch[1:1] = [pltpu.VMEM_SHARED((NP, CW), jnp.float32)]
        scratch[6:6] = [pltpu.VMEM((CH, CW), jnp.float32),
                        pltpu.VMEM((RPT, CW), jnp.float32)]

    @functools.partial(
        pl.kernel,
        out_type=tuple(out_type),
        mesh=plsc.VectorSubcoreMesh(core_axis_name="c", subcore_axis_name="s",
                                    num_cores=NC, num_subcores=NS),
        scratch_types=tuple(scratch),
        compiler_params=pltpu.CompilerParams(use_tc_tiling_on_sc=False),
    )
    def _sc_aggregate(*refs):
        _agg_body(with_cnt, *refs)

    return _sc_aggregate


BR = 1000


def _combine_body(relu, split_in, split_out,
                  x_ref, agg_ref, cnt_ref, wl_ref, wr_ref, b_ref, o_ref):
    cnt = cnt_ref[0, :, 0:1] + cnt_ref[1, :, 0:1]
    inv = 1.0 / jnp.maximum(cnt, 1.0)
    mean = jnp.concatenate([agg_ref[0], agg_ref[1]], axis=1) * inv
    xb = (jnp.concatenate([x_ref[0], x_ref[1]], axis=1) if split_in
          else x_ref[...])
    acc = (jnp.dot(mean, wl_ref[...], preferred_element_type=jnp.float32)
           + jnp.dot(xb, wr_ref[...], preferred_element_type=jnp.float32)
           + b_ref[...])
    if relu:
        acc = jnp.maximum(acc, 0.0)
    if split_out:
        o_ref[0] = acc[:, :HD]
        o_ref[1] = acc[:, HD:]
    else:
        o_ref[...] = acc


def _tc_combine(x, agg, cnt, Wl, Wr, b, relu, split_in, split_out):
    x_spec = (pl.BlockSpec((NC, BR, HD), lambda i: (0, i, 0)) if split_in
              else pl.BlockSpec((BR, D), lambda i: (i, 0)))
    if split_out:
        out_spec = pl.BlockSpec((NC, BR, HD), lambda i: (0, i, 0))
        out_shape = jax.ShapeDtypeStruct((NC, N, HD), jnp.float32)
    else:
        out_spec = pl.BlockSpec((BR, D), lambda i: (i, 0))
        out_shape = jax.ShapeDtypeStruct((N, D), jnp.float32)
    return pl.pallas_call(
        functools.partial(_combine_body, relu, split_in, split_out),
        grid=(N // BR,),
        in_specs=[
            x_spec,
            pl.BlockSpec((NC, BR, HD), lambda i: (0, i, 0)),
            pl.BlockSpec((NC, BR, CW), lambda i: (0, i, 0)),
            pl.BlockSpec((D, D), lambda i: (0, 0)),
            pl.BlockSpec((D, D), lambda i: (0, 0)),
            pl.BlockSpec((1, D), lambda i: (0, 0)),
        ],
        out_specs=out_spec,
        out_shape=out_shape,
    )(x, agg, cnt, Wl, Wr, b.reshape(1, D))


def kernel(x, edge_index, Wl1, Wr1, b1, Wl2, Wr2, b2):
    src = edge_index[0].astype(jnp.int32)
    dst = edge_index[1].astype(jnp.int32)
    srcp = jnp.concatenate(
        [src, jnp.zeros((PE - E,), jnp.int32)]).reshape(NCHT, CH)
    dstp = jnp.concatenate(
        [dst, jnp.full((PE - E,), TRASH, jnp.int32)]).reshape(NCHT, CH)
    x01 = jnp.stack([x[:, :HD], x[:, HD:]])
    agg1, cnt = _make_sc_aggregate(True)(x01, srcp, dstp)
    h01 = _tc_combine(x, agg1, cnt, Wl1, Wr1, b1,
                      relu=True, split_in=False, split_out=True)
    agg2, = _make_sc_aggregate(False)(h01, srcp, dstp)
    return _tc_combine(h01, agg2, cnt, Wl2, Wr2, b2,
                       relu=False, split_in=True, split_out=False)


# speedup vs baseline: 1.2177x; 1.1405x over previous
import functools

import jax
import jax.numpy as jnp
from jax import lax
from jax.experimental import pallas as pl
from jax.experimental.pallas import tpu as pltpu
from jax.experimental.pallas import tpu_sc as plsc

N = 10000
D = 128
HD = D // 2
E = 320000
NC = 2
NS = 16
CH = 128
CPT = 160
QUADS = CPT // 4
NCHT = NS * CPT
PE = NCHT * CH
NP = 10240
TRASH = NP - 1
RPT = NP // NS
ZCH = 128
CW = 16


def _agg_body(with_cnt, x01_hbm, src_hbm, dst_hbm, *refs):
    if with_cnt:
        (agg_hbm, cnt_hbm, acc_sh, r0, r1, r2, r3, src_v, dst_v,
         cnt_v, g0, g1, g2, g3, s0, s1, s2, s3) = refs
    else:
        (agg_hbm, acc_sh, r0, r1, r2, r3, src_v, dst_v,
         g0, g1, g2, g3, s0, s1, s2, s3) = refs
    rows = [r0, r1, r2, r3]
    gsem = [g0, g1, g2, g3]
    ssem = [s0, s1, s2, s3]
    c = lax.axis_index("c")
    s = lax.axis_index("s")
    cbase = s * CPT

    z16 = jnp.zeros((16,), jnp.float32)
    one16 = jnp.ones((16,), jnp.float32)

    def _zero_rows(r, _):
        for l in range(HD // 16):
            rows[0][r, pl.ds(l * 16, 16)] = z16
        return 0
    lax.fori_loop(0, ZCH, _zero_rows, 0)

    for q in range(RPT // ZCH):
        pltpu.sync_copy(rows[0].at[pl.ds(0, ZCH)],
                        acc_sh.at[pl.ds(s * RPT + q * ZCH, ZCH)])

    if with_cnt:
        def _zero_cnt(r, _):
            cnt_v[pl.ds(r * 16, 16)] = z16
            return 0
        lax.fori_loop(0, NP // 16, _zero_cnt, 0)

    pltpu.sync_copy(src_hbm.at[pl.ds(cbase, CPT)], src_v)
    pltpu.sync_copy(dst_hbm.at[pl.ds(cbase, CPT)], dst_v)

    plsc.subcore_barrier()

    table = x01_hbm.at[c]

    pltpu.async_copy(table.at[src_v.at[0]], rows[0], gsem[0])
    pltpu.async_copy(table.at[src_v.at[1]], rows[1], gsem[1])

    def _stage(q, i):
        ck = 4 * q + i
        j = i % 4
        jn = (i + 2) % 4
        pltpu.make_async_copy(table.at[src_v.at[ck]], rows[j],
                              gsem[j]).wait()
        pltpu.async_copy(rows[j], acc_sh.at[dst_v.at[ck]], ssem[j],
                         add=True)
        if with_cnt:
            @pl.when(c == 0)
            def _():
                for l in range(CH // 16):
                    idx = dst_v[ck, pl.ds(l * 16, 16)]
                    plsc.addupdate_scatter(cnt_v, [idx], one16)

        def _refill():
            pltpu.make_async_copy(rows[jn], acc_sh.at[dst_v.at[ck]],
                                  ssem[jn]).wait()
            pltpu.async_copy(table.at[src_v.at[ck + 2]], rows[jn], gsem[jn])

        if i < 2:
            @pl.when(q > 0)
            def _():
                _refill()

            @pl.when(q == 0)
            def _():
                pltpu.async_copy(table.at[src_v.at[ck + 2]], rows[jn],
                                 gsem[jn])
        else:
            @pl.when(q < QUADS - 1)
            def _():
                _refill()

            @pl.when(q == QUADS - 1)
            def _():
                pltpu.make_async_copy(rows[jn], acc_sh.at[dst_v.at[ck]],
                                      ssem[jn]).wait()

    def _edge_quad(q, _):
        for i in range(4):
            _stage(q, i)
        return 0
    lax.fori_loop(0, QUADS, _edge_quad, 0)

    pltpu.make_async_copy(rows[2], acc_sh.at[dst_v.at[CPT - 2]],
                          ssem[2]).wait()
    pltpu.make_async_copy(rows[3], acc_sh.at[dst_v.at[CPT - 1]],
                          ssem[3]).wait()

    plsc.subcore_barrier()

    pltpu.sync_copy(acc_sh.at[pl.ds(s * RPT, RPT)],
                    agg_hbm.at[c, pl.ds(s * RPT, RPT)])
    if with_cnt:
        @pl.when(c == 0)
        def _():
            pltpu.sync_copy(cnt_v, cnt_hbm.at[s])


@functools.lru_cache(maxsize=None)
def _make_sc_aggregate(with_cnt):
    out_type = [jax.ShapeDtypeStruct((NC, NP, HD), jnp.float32)]
    scratch = [
        pltpu.VMEM_SHARED((NP, HD), jnp.float32),
        pltpu.VMEM((CH, HD), jnp.float32),
        pltpu.VMEM((CH, HD), jnp.float32),
        pltpu.VMEM((CH, HD), jnp.float32),
        pltpu.VMEM((CH, HD), jnp.float32),
        pltpu.VMEM((CPT, CH), jnp.int32),
        pltpu.VMEM((CPT, CH), jnp.int32),
        pltpu.SemaphoreType.DMA,
        pltpu.SemaphoreType.DMA,
        pltpu.SemaphoreType.DMA,
        pltpu.SemaphoreType.DMA,
        pltpu.SemaphoreType.DMA,
        pltpu.SemaphoreType.DMA,
        pltpu.SemaphoreType.DMA,
        pltpu.SemaphoreType.DMA,
    ]
    if with_cnt:
        out_type.append(jax.ShapeDtypeStruct((NS, NP), jnp.float32))
        scratch[7:7] = [pltpu.VMEM((NP,), jnp.float32)]

    @functools.partial(
        pl.kernel,
        out_type=tuple(out_type),
        mesh=plsc.VectorSubcoreMesh(core_axis_name="c", subcore_axis_name="s",
                                    num_cores=NC, num_subcores=NS),
        scratch_types=tuple(scratch),
        compiler_params=pltpu.CompilerParams(use_tc_tiling_on_sc=False,
                                             needs_layout_passes=False),
    )
    def _sc_aggregate(*refs):
        _agg_body(with_cnt, *refs)

    return _sc_aggregate


BR = 1024


def _combine_body(relu, split_in, split_out,
                  x_ref, agg_ref, cnt_ref, wl_ref, wr_ref, b_ref, o_ref):
    i = pl.program_id(0)
    cnt = jnp.sum(cnt_ref[:, pl.ds(i * BR, BR)], axis=0)[:, None]
    inv = 1.0 / jnp.maximum(cnt, 1.0)
    mean = jnp.concatenate([agg_ref[0], agg_ref[1]], axis=1) * inv
    xb = (jnp.concatenate([x_ref[0], x_ref[1]], axis=1) if split_in
          else x_ref[...])
    acc = (jnp.dot(mean, wl_ref[...], preferred_element_type=jnp.float32)
           + jnp.dot(xb, wr_ref[...], preferred_element_type=jnp.float32)
           + b_ref[...])
    if relu:
        acc = jnp.maximum(acc, 0.0)
    if split_out:
        o_ref[0] = acc[:, :HD]
        o_ref[1] = acc[:, HD:]
    else:
        o_ref[...] = acc


def _tc_combine(x, agg, cnt, Wl, Wr, b, relu, split_in, split_out):
    x_spec = (pl.BlockSpec((NC, BR, HD), lambda i: (0, i, 0)) if split_in
              else pl.BlockSpec((BR, D), lambda i: (i, 0)))
    if split_out:
        out_spec = pl.BlockSpec((NC, BR, HD), lambda i: (0, i, 0))
        out_shape = jax.ShapeDtypeStruct((NC, N, HD), jnp.float32)
    else:
        out_spec = pl.BlockSpec((BR, D), lambda i: (i, 0))
        out_shape = jax.ShapeDtypeStruct((N, D), jnp.float32)
    return pl.pallas_call(
        functools.partial(_combine_body, relu, split_in, split_out),
        grid=((N + BR - 1) // BR,),
        in_specs=[
            x_spec,
            pl.BlockSpec((NC, BR, HD), lambda i: (0, i, 0)),
            pl.BlockSpec((NS, NP), lambda i: (0, 0)),
            pl.BlockSpec((D, D), lambda i: (0, 0)),
            pl.BlockSpec((D, D), lambda i: (0, 0)),
            pl.BlockSpec((1, D), lambda i: (0, 0)),
        ],
        out_specs=out_spec,
        out_shape=out_shape,
    )(x, agg, cnt, Wl, Wr, b.reshape(1, D))


def kernel(x, edge_index, Wl1, Wr1, b1, Wl2, Wr2, b2):
    src = edge_index[0].astype(jnp.int32)
    dst = edge_index[1].astype(jnp.int32)
    srcp = jnp.concatenate(
        [src, jnp.zeros((PE - E,), jnp.int32)]).reshape(NCHT, CH)
    dstp = jnp.concatenate(
        [dst, jnp.full((PE - E,), TRASH, jnp.int32)]).reshape(NCHT, CH)
    x01 = jnp.stack([x[:, :HD], x[:, HD:]])
    agg1, cnt = _make_sc_aggregate(True)(x01, srcp, dstp)
    h01 = _tc_combine(x, agg1, cnt, Wl1, Wr1, b1,
                      relu=True, split_in=False, split_out=True)
    agg2, = _make_sc_aggregate(False)(h01, srcp, dstp)
    return _tc_combine(h01, agg2, cnt, Wl2, Wr2, b2,
                       relu=False, split_in=True, split_out=False)


# speedup vs baseline: 1.2318x; 1.0116x over previous
import functools

import jax
import jax.numpy as jnp
from jax import lax
from jax.experimental import pallas as pl
from jax.experimental.pallas import tpu as pltpu
from jax.experimental.pallas import tpu_sc as plsc

N = 10000
D = 128
HD = D // 2
E = 320000
NC = 2
NS = 16
CH = 128
CPT = 160
PAIRS = CPT // 2
NCHT = NS * CPT
PE = NCHT * CH
NP = 10240
TRASH = NP - 1
RPT = NP // NS
ZCH = 128
CW = 16


def _agg_body(with_cnt, x01_hbm, src_hbm, dst_hbm, *refs):
    if with_cnt:
        (agg_hbm, cnt_hbm, acc_sh, r0, r1, src_v, dst_v,
         cnt_v, g0, g1) = refs
    else:
        (agg_hbm, acc_sh, r0, r1, src_v, dst_v, g0, g1) = refs
    rows = [r0, r1]
    gsem = [g0, g1]
    c = lax.axis_index("c")
    s = lax.axis_index("s")
    cbase = s * CPT

    z16 = jnp.zeros((16,), jnp.float32)
    one16 = jnp.ones((16,), jnp.float32)

    def _zero_rows(r, _):
        for l in range(HD // 16):
            rows[0][r, pl.ds(l * 16, 16)] = z16
        return 0
    lax.fori_loop(0, ZCH, _zero_rows, 0)

    for q in range(RPT // ZCH):
        pltpu.sync_copy(rows[0].at[pl.ds(0, ZCH)],
                        acc_sh.at[pl.ds(s * RPT + q * ZCH, ZCH)])

    if with_cnt:
        def _zero_cnt(r, _):
            cnt_v[pl.ds(r * 16, 16)] = z16
            return 0
        lax.fori_loop(0, NP // 16, _zero_cnt, 0)

    pltpu.sync_copy(src_hbm.at[pl.ds(cbase, CPT)], src_v)
    pltpu.sync_copy(dst_hbm.at[pl.ds(cbase, CPT)], dst_v)

    plsc.subcore_barrier()

    table = x01_hbm.at[c]

    pltpu.async_copy(table.at[src_v.at[0]], rows[0], gsem[0])
    pltpu.async_copy(table.at[src_v.at[1]], rows[1], gsem[1])

    def _half(k, ck, rows_j, sem_j):
        pltpu.make_async_copy(table.at[src_v.at[ck]], rows_j, sem_j).wait()
        pltpu.sync_copy(rows_j, acc_sh.at[dst_v.at[ck]], add=True)

        @pl.when(k < PAIRS - 1)
        def _():
            pltpu.async_copy(table.at[src_v.at[ck + 2]], rows_j, sem_j)
        if with_cnt:
            @pl.when(c == 0)
            def _():
                for l in range(CH // 16):
                    idx = dst_v[ck, pl.ds(l * 16, 16)]
                    plsc.addupdate_scatter(cnt_v, [idx], one16)

    def _edge_pair(k, _):
        _half(k, 2 * k, rows[0], gsem[0])
        _half(k, 2 * k + 1, rows[1], gsem[1])
        return 0
    lax.fori_loop(0, PAIRS, _edge_pair, 0)

    plsc.subcore_barrier()

    pltpu.sync_copy(acc_sh.at[pl.ds(s * RPT, RPT)],
                    agg_hbm.at[c, pl.ds(s * RPT, RPT)])
    if with_cnt:
        @pl.when(c == 0)
        def _():
            pltpu.sync_copy(cnt_v, cnt_hbm.at[s])


@functools.lru_cache(maxsize=None)
def _make_sc_aggregate(with_cnt):
    out_type = [jax.ShapeDtypeStruct((NC, NP, HD), jnp.float32)]
    scratch = [
        pltpu.VMEM_SHARED((NP, HD), jnp.float32),
        pltpu.VMEM((CH, HD), jnp.float32),
        pltpu.VMEM((CH, HD), jnp.float32),
        pltpu.VMEM((CPT, CH), jnp.int32),
        pltpu.VMEM((CPT, CH), jnp.int32),
        pltpu.SemaphoreType.DMA,
        pltpu.SemaphoreType.DMA,
    ]
    if with_cnt:
        out_type.append(jax.ShapeDtypeStruct((NS, NP), jnp.float32))
        scratch[5:5] = [pltpu.VMEM((NP,), jnp.float32)]

    @functools.partial(
        pl.kernel,
        out_type=tuple(out_type),
        mesh=plsc.VectorSubcoreMesh(core_axis_name="c", subcore_axis_name="s",
                                    num_cores=NC, num_subcores=NS),
        scratch_types=tuple(scratch),
        compiler_params=pltpu.CompilerParams(use_tc_tiling_on_sc=False,
                                             needs_layout_passes=False),
    )
    def _sc_aggregate(*refs):
        _agg_body(with_cnt, *refs)

    return _sc_aggregate


BR = 1024


def _combine_body(relu, split_in, split_out,
                  x_ref, agg_ref, cnt_ref, wl_ref, wr_ref, b_ref, o_ref):
    i = pl.program_id(0)
    cnt = jnp.sum(cnt_ref[:, pl.ds(i * BR, BR)], axis=0)[:, None]
    inv = 1.0 / jnp.maximum(cnt, 1.0)
    mean = jnp.concatenate([agg_ref[0], agg_ref[1]], axis=1) * inv
    xb = (jnp.concatenate([x_ref[0], x_ref[1]], axis=1) if split_in
          else x_ref[...])
    acc = (jnp.dot(mean, wl_ref[...], preferred_element_type=jnp.float32)
           + jnp.dot(xb, wr_ref[...], preferred_element_type=jnp.float32)
           + b_ref[...])
    if relu:
        acc = jnp.maximum(acc, 0.0)
    if split_out:
        o_ref[0] = acc[:, :HD]
        o_ref[1] = acc[:, HD:]
    else:
        o_ref[...] = acc


def _tc_combine(x, agg, cnt, Wl, Wr, b, relu, split_in, split_out):
    x_spec = (pl.BlockSpec((NC, BR, HD), lambda i: (0, i, 0)) if split_in
              else pl.BlockSpec((BR, D), lambda i: (i, 0)))
    if split_out:
        out_spec = pl.BlockSpec((NC, BR, HD), lambda i: (0, i, 0))
        out_shape = jax.ShapeDtypeStruct((NC, N, HD), jnp.float32)
    else:
        out_spec = pl.BlockSpec((BR, D), lambda i: (i, 0))
        out_shape = jax.ShapeDtypeStruct((N, D), jnp.float32)
    return pl.pallas_call(
        functools.partial(_combine_body, relu, split_in, split_out),
        grid=((N + BR - 1) // BR,),
        in_specs=[
            x_spec,
            pl.BlockSpec((NC, BR, HD), lambda i: (0, i, 0)),
            pl.BlockSpec((NS, NP), lambda i: (0, 0)),
            pl.BlockSpec((D, D), lambda i: (0, 0)),
            pl.BlockSpec((D, D), lambda i: (0, 0)),
            pl.BlockSpec((1, D), lambda i: (0, 0)),
        ],
        out_specs=out_spec,
        out_shape=out_shape,
    )(x, agg, cnt, Wl, Wr, b.reshape(1, D))


def kernel(x, edge_index, Wl1, Wr1, b1, Wl2, Wr2, b2):
    src = edge_index[0].astype(jnp.int32)
    dst = edge_index[1].astype(jnp.int32)
    srcp = jnp.concatenate(
        [src, jnp.zeros((PE - E,), jnp.int32)]).reshape(NCHT, CH)
    dstp = jnp.concatenate(
        [dst, jnp.full((PE - E,), TRASH, jnp.int32)]).reshape(NCHT, CH)
    x01 = jnp.stack([x[:, :HD], x[:, HD:]])
    agg1, cnt = _make_sc_aggregate(True)(x01, srcp, dstp)
    h01 = _tc_combine(x, agg1, cnt, Wl1, Wr1, b1,
                      relu=True, split_in=False, split_out=True)
    agg2, = _make_sc_aggregate(False)(h01, srcp, dstp)
    return _tc_combine(h01, agg2, cnt, Wl2, Wr2, b2,
                       relu=False, split_in=True, split_out=False)


# speedup vs baseline: 1.2428x; 1.0090x over previous
import functools

import jax
import jax.numpy as jnp
from jax import lax
from jax.experimental import pallas as pl
from jax.experimental.pallas import tpu as pltpu
from jax.experimental.pallas import tpu_sc as plsc

N = 10000
D = 128
HD = D // 2
E = 320000
NC = 2
NS = 16
CH = 128
CPT = 160
PAIRS = CPT // 2
NCHT = NS * CPT
PE = NCHT * CH
NP = 10240
TRASH = NP - 1
RPT = NP // NS
ZCH = 128
CW = 16


def _agg_body(with_cnt, x01_hbm, src_hbm, dst_hbm, *refs):
    if with_cnt:
        (agg_hbm, cnt_hbm, acc_sh, r0, r1, src_v, dst_v,
         cnt_v, g0, g1) = refs
    else:
        (agg_hbm, acc_sh, r0, r1, src_v, dst_v, g0, g1) = refs
    rows = [r0, r1]
    gsem = [g0, g1]
    c = lax.axis_index("c")
    s = lax.axis_index("s")
    cbase = s * CPT

    z16 = jnp.zeros((16,), jnp.float32)
    one16 = jnp.ones((16,), jnp.float32)

    def _zero_rows(r, _):
        for l in range(HD // 16):
            rows[0][r, pl.ds(l * 16, 16)] = z16
        return 0
    lax.fori_loop(0, ZCH, _zero_rows, 0)

    for q in range(RPT // ZCH):
        pltpu.sync_copy(rows[0].at[pl.ds(0, ZCH)],
                        acc_sh.at[pl.ds(s * RPT + q * ZCH, ZCH)])

    if with_cnt:
        def _zero_cnt(r, _):
            cnt_v[pl.ds(r * 16, 16)] = z16
            return 0
        lax.fori_loop(0, NP // 16, _zero_cnt, 0)

    pltpu.sync_copy(src_hbm.at[pl.ds(cbase, CPT)], src_v)
    pltpu.sync_copy(dst_hbm.at[pl.ds(cbase, CPT)], dst_v)

    plsc.subcore_barrier()

    table = x01_hbm.at[c]

    pltpu.async_copy(table.at[src_v.at[0]], rows[0], gsem[0])
    pltpu.async_copy(table.at[src_v.at[1]], rows[1], gsem[1])

    def _half(k, ck, rows_j, sem_j, cnt_core):
        pltpu.make_async_copy(table.at[src_v.at[ck]], rows_j, sem_j).wait()
        pltpu.sync_copy(rows_j, acc_sh.at[dst_v.at[ck]], add=True)

        @pl.when(k < PAIRS - 1)
        def _():
            pltpu.async_copy(table.at[src_v.at[ck + 2]], rows_j, sem_j)
        if with_cnt:
            @pl.when(c == cnt_core)
            def _():
                for l in range(CH // 16):
                    idx = dst_v[ck, pl.ds(l * 16, 16)]
                    plsc.addupdate_scatter(cnt_v, [idx], one16)

    def _edge_pair(k, _):
        _half(k, 2 * k, rows[0], gsem[0], 0)
        _half(k, 2 * k + 1, rows[1], gsem[1], 1)
        return 0
    lax.fori_loop(0, PAIRS, _edge_pair, 0)

    plsc.subcore_barrier()

    pltpu.sync_copy(acc_sh.at[pl.ds(s * RPT, RPT)],
                    agg_hbm.at[c, pl.ds(s * RPT, RPT)])
    if with_cnt:
        pltpu.sync_copy(cnt_v, cnt_hbm.at[c, s])


@functools.lru_cache(maxsize=None)
def _make_sc_aggregate(with_cnt):
    out_type = [jax.ShapeDtypeStruct((NC, NP, HD), jnp.float32)]
    scratch = [
        pltpu.VMEM_SHARED((NP, HD), jnp.float32),
        pltpu.VMEM((CH, HD), jnp.float32),
        pltpu.VMEM((CH, HD), jnp.float32),
        pltpu.VMEM((CPT, CH), jnp.int32),
        pltpu.VMEM((CPT, CH), jnp.int32),
        pltpu.SemaphoreType.DMA,
        pltpu.SemaphoreType.DMA,
    ]
    if with_cnt:
        out_type.append(jax.ShapeDtypeStruct((NC, NS, NP), jnp.float32))
        scratch[5:5] = [pltpu.VMEM((NP,), jnp.float32)]

    @functools.partial(
        pl.kernel,
        out_type=tuple(out_type),
        mesh=plsc.VectorSubcoreMesh(core_axis_name="c", subcore_axis_name="s",
                                    num_cores=NC, num_subcores=NS),
        scratch_types=tuple(scratch),
        compiler_params=pltpu.CompilerParams(use_tc_tiling_on_sc=False,
                                             needs_layout_passes=False),
    )
    def _sc_aggregate(*refs):
        _agg_body(with_cnt, *refs)

    return _sc_aggregate


BR = 1024


def _combine_body(relu, split_in, split_out,
                  x_ref, agg_ref, cnt_ref, wl_ref, wr_ref, b_ref, o_ref):
    i = pl.program_id(0)
    cnt = jnp.sum(cnt_ref[:, :, pl.ds(i * BR, BR)], axis=(0, 1))[:, None]
    inv = 1.0 / jnp.maximum(cnt, 1.0)
    mean = jnp.concatenate([agg_ref[0], agg_ref[1]], axis=1) * inv
    xb = (jnp.concatenate([x_ref[0], x_ref[1]], axis=1) if split_in
          else x_ref[...])
    acc = (jnp.dot(mean, wl_ref[...], preferred_element_type=jnp.float32)
           + jnp.dot(xb, wr_ref[...], preferred_element_type=jnp.float32)
           + b_ref[...])
    if relu:
        acc = jnp.maximum(acc, 0.0)
    if split_out:
        o_ref[0] = acc[:, :HD]
        o_ref[1] = acc[:, HD:]
    else:
        o_ref[...] = acc


def _tc_combine(x, agg, cnt, Wl, Wr, b, relu, split_in, split_out):
    x_spec = (pl.BlockSpec((NC, BR, HD), lambda i: (0, i, 0)) if split_in
              else pl.BlockSpec((BR, D), lambda i: (i, 0)))
    if split_out:
        out_spec = pl.BlockSpec((NC, BR, HD), lambda i: (0, i, 0))
        out_shape = jax.ShapeDtypeStruct((NC, N, HD), jnp.float32)
    else:
        out_spec = pl.BlockSpec((BR, D), lambda i: (i, 0))
        out_shape = jax.ShapeDtypeStruct((N, D), jnp.float32)
    return pl.pallas_call(
        functools.partial(_combine_body, relu, split_in, split_out),
        grid=((N + BR - 1) // BR,),
        in_specs=[
            x_spec,
            pl.BlockSpec((NC, BR, HD), lambda i: (0, i, 0)),
            pl.BlockSpec((NC, NS, NP), lambda i: (0, 0, 0)),
            pl.BlockSpec((D, D), lambda i: (0, 0)),
            pl.BlockSpec((D, D), lambda i: (0, 0)),
            pl.BlockSpec((1, D), lambda i: (0, 0)),
        ],
        out_specs=out_spec,
        out_shape=out_shape,
    )(x, agg, cnt, Wl, Wr, b.reshape(1, D))


def kernel(x, edge_index, Wl1, Wr1, b1, Wl2, Wr2, b2):
    src = edge_index[0].astype(jnp.int32)
    dst = edge_index[1].astype(jnp.int32)
    srcp = jnp.concatenate(
        [src, jnp.zeros((PE - E,), jnp.int32)]).reshape(NCHT, CH)
    dstp = jnp.concatenate(
        [dst, jnp.full((PE - E,), TRASH, jnp.int32)]).reshape(NCHT, CH)
    x01 = jnp.stack([x[:, :HD], x[:, HD:]])
    agg1, cnt = _make_sc_aggregate(True)(x01, srcp, dstp)
    h01 = _tc_combine(x, agg1, cnt, Wl1, Wr1, b1,
                      relu=True, split_in=False, split_out=True)
    agg2, = _make_sc_aggregate(False)(h01, srcp, dstp)
    return _tc_combine(h01, agg2, cnt, Wl2, Wr2, b2,
                       relu=False, split_in=True, split_out=False)
